# Initial kernel scaffold; baseline (speedup 1.0000x reference)
import jax
import jax.numpy as jnp
from jax.experimental import pallas as pl


def kernel(x, edge_index_rel0, edge_index_rel1, batch, W0_rel0, b0_rel0, W0_rel1, b0_rel1, W1_rel0, b1_rel0, W1_rel1, b1_rel1):
    raise NotImplementedError("write your pallas kernel here")


# speedup vs baseline: 1.5051x; 1.5051x over previous
import functools

import jax
import jax.numpy as jnp
from jax.experimental import pallas as pl
from jax.experimental.pallas import tpu as pltpu


def _mm_scale_body(x_ref, w_ref, d_ref, o_ref):
    o_ref[...] = jnp.dot(x_ref[...], w_ref[...],
                         preferred_element_type=jnp.float32) * d_ref[...]


def _mm1_body(a0_ref, a1_ref, d0_ref, d1_ref, dr_ref, w_ref, b_ref, o_ref):
    h = jnp.maximum(a0_ref[...] * d0_ref[...] + a1_ref[...] * d1_ref[...]
                    + b_ref[...], 0.0)
    o_ref[...] = jnp.dot(h, w_ref[...],
                         preferred_element_type=jnp.float32) * dr_ref[...]


def _deg_body(c_ref, o_ref, *, block_e, n):
    @pl.when(pl.program_id(0) == 0)
    def _():
        o_ref[...] = jnp.zeros((n, 1), jnp.float32)

    def body(j, carry):
        c = c_ref[0, 0, j]
        o_ref[pl.ds(c, 1), :] = o_ref[pl.ds(c, 1), :] + 1.0
        return carry

    jax.lax.fori_loop(0, block_e, body, 0)


def _agg_body(ms_ref, r_ref, c_ref, o_ref, *, block_e):
    @pl.when(pl.program_id(0) == 0)
    def _():
        o_ref[...] = ms_ref[...]

    def body(j, carry):
        r = r_ref[0, 0, j]
        c = c_ref[0, 0, j]
        o_ref[pl.ds(c, 1), :] = o_ref[pl.ds(c, 1), :] + ms_ref[pl.ds(r, 1), :]
        return carry

    jax.lax.fori_loop(0, block_e, body, 0)


def _pool_body(a0_ref, a1_ref, d0_ref, d1_ref, b_ref, batch_ref, o_ref,
               cnt_ref, *, g, block_n, n_blocks):
    @pl.when(pl.program_id(0) == 0)
    def _():
        o_ref[...] = jnp.zeros_like(o_ref)
        cnt_ref[...] = jnp.zeros_like(cnt_ref)

    final = (a0_ref[...] * d0_ref[...] + a1_ref[...] * d1_ref[...]
             + b_ref[...])
    ids = jax.lax.broadcasted_iota(jnp.int32, (g, block_n), 0)
    onehot = (ids == batch_ref[0, 0, :][None, :]).astype(jnp.float32)
    o_ref[...] += jnp.dot(onehot, final, preferred_element_type=jnp.float32)
    cnt_ref[...] += jnp.broadcast_to(
        jnp.sum(onehot, axis=1, keepdims=True), cnt_ref.shape)

    @pl.when(pl.program_id(0) == n_blocks - 1)
    def _():
        o_ref[...] = o_ref[...] / jnp.maximum(cnt_ref[...], 1.0)


def _pick_block(total, target):
    for b in range(min(target, total), 0, -1):
        if total % b == 0:
            return b
    return total


def kernel(x, edge_index_rel0, edge_index_rel1, batch,
           W0_rel0, b0_rel0, W0_rel1, b0_rel1,
           W1_rel0, b1_rel0, W1_rel1, b1_rel1):
    f32 = jnp.float32
    n, d_in = x.shape
    d_hid = W0_rel0.shape[1]
    d_out = W1_rel0.shape[1]
    e = edge_index_rel0.shape[1]
    g = 64

    block_n = _pick_block(n, 2000)
    n_nb = n // block_n
    block_e = _pick_block(e, 4000)
    e_nb = e // block_e

    rows0 = edge_index_rel0[0].astype(jnp.int32).reshape(e_nb, 1, block_e)
    cols0 = edge_index_rel0[1].astype(jnp.int32).reshape(e_nb, 1, block_e)
    rows1 = edge_index_rel1[0].astype(jnp.int32).reshape(e_nb, 1, block_e)
    cols1 = edge_index_rel1[1].astype(jnp.int32).reshape(e_nb, 1, block_e)
    batch3 = batch.astype(jnp.int32).reshape(n_nb, 1, block_n)

    smem_idx = pl.BlockSpec((1, 1, block_e), lambda i: (i, 0, 0),
                            memory_space=pltpu.SMEM)

    deg_call = pl.pallas_call(
        functools.partial(_deg_body, block_e=block_e, n=n),
        grid=(e_nb,),
        in_specs=[smem_idx],
        out_specs=pl.BlockSpec((n, 1), lambda i: (0, 0)),
        out_shape=jax.ShapeDtypeStruct((n, 1), f32),
    )
    dinv0 = jax.lax.rsqrt(deg_call(cols0) + 1.0)
    dinv1 = jax.lax.rsqrt(deg_call(cols1) + 1.0)

    def mm_scale(xv, w, dinv):
        return pl.pallas_call(
            _mm_scale_body,
            grid=(n_nb,),
            in_specs=[
                pl.BlockSpec((block_n, d_in), lambda i: (i, 0)),
                pl.BlockSpec((d_in, d_hid), lambda i: (0, 0)),
                pl.BlockSpec((block_n, 1), lambda i: (i, 0)),
            ],
            out_specs=pl.BlockSpec((block_n, d_hid), lambda i: (i, 0)),
            out_shape=jax.ShapeDtypeStruct((n, d_hid), f32),
        )(xv, w, dinv)

    def agg(ms, rows3, cols3):
        return pl.pallas_call(
            functools.partial(_agg_body, block_e=block_e),
            grid=(e_nb,),
            in_specs=[
                pl.BlockSpec((n, d_hid), lambda i: (0, 0)),
                smem_idx,
                smem_idx,
            ],
            out_specs=pl.BlockSpec((n, d_hid), lambda i: (0, 0)),
            out_shape=jax.ShapeDtypeStruct((n, d_hid), f32),
        )(ms, rows3, cols3)

    ms00 = mm_scale(x, W0_rel0, dinv0)
    ms01 = mm_scale(x, W0_rel1, dinv1)
    agg00 = agg(ms00, rows0, cols0)
    agg01 = agg(ms01, rows1, cols1)

    bsum0 = (b0_rel0 + b0_rel1).reshape(1, d_hid)

    def mm1(dr, w):
        return pl.pallas_call(
            _mm1_body,
            grid=(n_nb,),
            in_specs=[
                pl.BlockSpec((block_n, d_hid), lambda i: (i, 0)),
                pl.BlockSpec((block_n, d_hid), lambda i: (i, 0)),
                pl.BlockSpec((block_n, 1), lambda i: (i, 0)),
                pl.BlockSpec((block_n, 1), lambda i: (i, 0)),
                pl.BlockSpec((block_n, 1), lambda i: (i, 0)),
                pl.BlockSpec((d_hid, d_out), lambda i: (0, 0)),
                pl.BlockSpec((1, d_hid), lambda i: (0, 0)),
            ],
            out_specs=pl.BlockSpec((block_n, d_out), lambda i: (i, 0)),
            out_shape=jax.ShapeDtypeStruct((n, d_out), f32),
        )(agg00, agg01, dinv0, dinv1, dr, w, bsum0)

    ms10 = mm1(dinv0, W1_rel0)
    ms11 = mm1(dinv1, W1_rel1)
    agg10 = agg(ms10, rows0, cols0)
    agg11 = agg(ms11, rows1, cols1)

    bsum1 = (b1_rel0 + b1_rel1).reshape(1, d_out)

    pooled = pl.pallas_call(
        functools.partial(_pool_body, g=g, block_n=block_n, n_blocks=n_nb),
        grid=(n_nb,),
        in_specs=[
            pl.BlockSpec((block_n, d_out), lambda i: (i, 0)),
            pl.BlockSpec((block_n, d_out), lambda i: (i, 0)),
            pl.BlockSpec((block_n, 1), lambda i: (i, 0)),
            pl.BlockSpec((block_n, 1), lambda i: (i, 0)),
            pl.BlockSpec((1, d_out), lambda i: (0, 0)),
            pl.BlockSpec((1, 1, block_n), lambda i: (i, 0, 0)),
        ],
        out_specs=pl.BlockSpec((g, d_out), lambda i: (0, 0)),
        out_shape=jax.ShapeDtypeStruct((g, d_out), f32),
        scratch_shapes=[pltpu.VMEM((g, d_out), f32)],
    )(agg10, agg11, dinv0, dinv1, bsum1, batch3)

    return pooled


# speedup vs baseline: 2.6685x; 1.7730x over previous
import functools

import jax
import jax.numpy as jnp
from jax.experimental import pallas as pl
from jax.experimental.pallas import tpu as pltpu


def _mm_scale_body(x_ref, w_ref, d_ref, o_ref):
    o_ref[...] = jnp.dot(x_ref[...], w_ref[...],
                         preferred_element_type=jnp.float32) * d_ref[...]


def _mm1_body(a0_ref, a1_ref, d0_ref, d1_ref, dr_ref, w_ref, b_ref, o_ref):
    h = jnp.maximum(a0_ref[...] * d0_ref[...] + a1_ref[...] * d1_ref[...]
                    + b_ref[...], 0.0)
    o_ref[...] = jnp.dot(h, w_ref[...],
                         preferred_element_type=jnp.float32) * dr_ref[...]


def _deg_body(c_ref, o_ref, *, block_e, n):
    @pl.when(pl.program_id(0) == 0)
    def _():
        o_ref[...] = jnp.zeros((n, 1), jnp.float32)

    def body(j, carry):
        for k in range(4):
            c = c_ref[0, 0, j * 4 + k]
            o_ref[pl.ds(c, 1), :] = o_ref[pl.ds(c, 1), :] + 1.0
        return carry

    jax.lax.fori_loop(0, block_e // 4, body, 0)


def _agg_body(ms_ref, r_ref, c_ref, o_ref, *, block_e):
    @pl.when(pl.program_id(0) == 0)
    def _():
        o_ref[...] = ms_ref[...]

    def body(j, carry):
        for k in range(4):
            r = r_ref[0, 0, j * 4 + k]
            c = c_ref[0, 0, j * 4 + k]
            o_ref[pl.ds(c, 1), :] = (o_ref[pl.ds(c, 1), :]
                                     + ms_ref[pl.ds(r, 1), :])
        return carry

    jax.lax.fori_loop(0, block_e // 4, body, 0)


def _pool_body(a0_ref, a1_ref, d0_ref, d1_ref, b_ref, batch_ref, o_ref,
               cnt_ref, *, g, block_n, n_blocks):
    @pl.when(pl.program_id(0) == 0)
    def _():
        o_ref[...] = jnp.zeros_like(o_ref)
        cnt_ref[...] = jnp.zeros_like(cnt_ref)

    final = (a0_ref[...] * d0_ref[...] + a1_ref[...] * d1_ref[...]
             + b_ref[...])
    ids = jax.lax.broadcasted_iota(jnp.int32, (g, block_n), 0)
    onehot = (ids == batch_ref[0, 0, :][None, :]).astype(jnp.float32)
    o_ref[...] += jnp.dot(onehot, final, preferred_element_type=jnp.float32)
    cnt_ref[...] += jnp.broadcast_to(
        jnp.sum(onehot, axis=1, keepdims=True), cnt_ref.shape)

    @pl.when(pl.program_id(0) == n_blocks - 1)
    def _():
        o_ref[...] = o_ref[...] / jnp.maximum(cnt_ref[...], 1.0)


def _pick_block(total, target):
    for b in range(min(target, total), 0, -1):
        if total % b == 0:
            return b
    return total


def kernel(x, edge_index_rel0, edge_index_rel1, batch,
           W0_rel0, b0_rel0, W0_rel1, b0_rel1,
           W1_rel0, b1_rel0, W1_rel1, b1_rel1):
    f32 = jnp.float32
    n, d_in = x.shape
    d_hid = W0_rel0.shape[1]
    d_out = W1_rel0.shape[1]
    e = edge_index_rel0.shape[1]
    g = 64

    block_n = _pick_block(n, 2000)
    n_nb = n // block_n
    block_e = _pick_block(e, 4000)
    e_nb = e // block_e

    rows0 = edge_index_rel0[0].astype(jnp.int32).reshape(e_nb, 1, block_e)
    cols0 = edge_index_rel0[1].astype(jnp.int32).reshape(e_nb, 1, block_e)
    rows1 = edge_index_rel1[0].astype(jnp.int32).reshape(e_nb, 1, block_e)
    cols1 = edge_index_rel1[1].astype(jnp.int32).reshape(e_nb, 1, block_e)
    batch3 = batch.astype(jnp.int32).reshape(n_nb, 1, block_n)

    smem_idx = pl.BlockSpec((1, 1, block_e), lambda i: (i, 0, 0),
                            memory_space=pltpu.SMEM)

    deg_call = pl.pallas_call(
        functools.partial(_deg_body, block_e=block_e, n=n),
        grid=(e_nb,),
        in_specs=[smem_idx],
        out_specs=pl.BlockSpec((n, 1), lambda i: (0, 0)),
        out_shape=jax.ShapeDtypeStruct((n, 1), f32),
    )
    dinv0 = jax.lax.rsqrt(deg_call(cols0) + 1.0)
    dinv1 = jax.lax.rsqrt(deg_call(cols1) + 1.0)

    def mm_scale(xv, w, dinv):
        return pl.pallas_call(
            _mm_scale_body,
            grid=(n_nb,),
            in_specs=[
                pl.BlockSpec((block_n, d_in), lambda i: (i, 0)),
                pl.BlockSpec((d_in, d_hid), lambda i: (0, 0)),
                pl.BlockSpec((block_n, 1), lambda i: (i, 0)),
            ],
            out_specs=pl.BlockSpec((block_n, d_hid), lambda i: (i, 0)),
            out_shape=jax.ShapeDtypeStruct((n, d_hid), f32),
        )(xv, w, dinv)

    def agg(ms, rows3, cols3):
        return pl.pallas_call(
            functools.partial(_agg_body, block_e=block_e),
            grid=(e_nb,),
            in_specs=[
                pl.BlockSpec((n, d_hid), lambda i: (0, 0)),
                smem_idx,
                smem_idx,
            ],
            out_specs=pl.BlockSpec((n, d_hid), lambda i: (0, 0)),
            out_shape=jax.ShapeDtypeStruct((n, d_hid), f32),
        )(ms, rows3, cols3)

    ms00 = mm_scale(x, W0_rel0, dinv0)
    ms01 = mm_scale(x, W0_rel1, dinv1)
    agg00 = agg(ms00, rows0, cols0)
    agg01 = agg(ms01, rows1, cols1)

    bsum0 = (b0_rel0 + b0_rel1).reshape(1, d_hid)

    def mm1(dr, w):
        return pl.pallas_call(
            _mm1_body,
            grid=(n_nb,),
            in_specs=[
                pl.BlockSpec((block_n, d_hid), lambda i: (i, 0)),
                pl.BlockSpec((block_n, d_hid), lambda i: (i, 0)),
                pl.BlockSpec((block_n, 1), lambda i: (i, 0)),
                pl.BlockSpec((block_n, 1), lambda i: (i, 0)),
                pl.BlockSpec((block_n, 1), lambda i: (i, 0)),
                pl.BlockSpec((d_hid, d_out), lambda i: (0, 0)),
                pl.BlockSpec((1, d_hid), lambda i: (0, 0)),
            ],
            out_specs=pl.BlockSpec((block_n, d_out), lambda i: (i, 0)),
            out_shape=jax.ShapeDtypeStruct((n, d_out), f32),
        )(agg00, agg01, dinv0, dinv1, dr, w, bsum0)

    ms10 = mm1(dinv0, W1_rel0)
    ms11 = mm1(dinv1, W1_rel1)
    agg10 = agg(ms10, rows0, cols0)
    agg11 = agg(ms11, rows1, cols1)

    bsum1 = (b1_rel0 + b1_rel1).reshape(1, d_out)

    pooled = pl.pallas_call(
        functools.partial(_pool_body, g=g, block_n=block_n, n_blocks=n_nb),
        grid=(n_nb,),
        in_specs=[
            pl.BlockSpec((block_n, d_out), lambda i: (i, 0)),
            pl.BlockSpec((block_n, d_out), lambda i: (i, 0)),
            pl.BlockSpec((block_n, 1), lambda i: (i, 0)),
            pl.BlockSpec((block_n, 1), lambda i: (i, 0)),
            pl.BlockSpec((1, d_out), lambda i: (0, 0)),
            pl.BlockSpec((1, 1, block_n), lambda i: (i, 0, 0)),
        ],
        out_specs=pl.BlockSpec((g, d_out), lambda i: (0, 0)),
        out_shape=jax.ShapeDtypeStruct((g, d_out), f32),
        scratch_shapes=[pltpu.VMEM((g, d_out), f32)],
    )(agg10, agg11, dinv0, dinv1, bsum1, batch3)

    return pooled


# speedup vs baseline: 2.8643x; 1.0734x over previous
import functools

import jax
import jax.numpy as jnp
from jax.experimental import pallas as pl
from jax.experimental.pallas import tpu as pltpu


def _mm_scale_body(x_ref, w_ref, d_ref, o_ref):
    o_ref[...] = jnp.dot(x_ref[...], w_ref[...],
                         preferred_element_type=jnp.float32) * d_ref[...]


def _mm1_body(a0_ref, a1_ref, d0_ref, d1_ref, dr_ref, w_ref, b_ref, o_ref):
    h = jnp.maximum(a0_ref[...] * d0_ref[...] + a1_ref[...] * d1_ref[...]
                    + b_ref[...], 0.0)
    o_ref[...] = jnp.dot(h, w_ref[...],
                         preferred_element_type=jnp.float32) * dr_ref[...]


def _deg_body(c_ref, o_ref, *, block_e, n):
    @pl.when(pl.program_id(0) == 0)
    def _():
        o_ref[...] = jnp.zeros((n, 1), jnp.float32)

    def body(j, carry):
        for k in range(8):
            c = c_ref[0, 0, j * 8 + k]
            o_ref[pl.ds(c, 1), :] = o_ref[pl.ds(c, 1), :] + 1.0
        return carry

    jax.lax.fori_loop(0, block_e // 8, body, 0)


def _agg_body(ms_ref, r_ref, c_ref, o_ref, *, block_e):
    @pl.when(pl.program_id(0) == 0)
    def _():
        o_ref[...] = ms_ref[...]

    def body(j, carry):
        for k in range(8):
            r = r_ref[0, 0, j * 8 + k]
            c = c_ref[0, 0, j * 8 + k]
            o_ref[pl.ds(c, 1), :] = (o_ref[pl.ds(c, 1), :]
                                     + ms_ref[pl.ds(r, 1), :])
        return carry

    jax.lax.fori_loop(0, block_e // 8, body, 0)


def _pool_body(a0_ref, a1_ref, d0_ref, d1_ref, b_ref, batch_ref, o_ref,
               cnt_ref, *, g, block_n, n_blocks):
    @pl.when(pl.program_id(0) == 0)
    def _():
        o_ref[...] = jnp.zeros_like(o_ref)
        cnt_ref[...] = jnp.zeros_like(cnt_ref)

    final = (a0_ref[...] * d0_ref[...] + a1_ref[...] * d1_ref[...]
             + b_ref[...])
    ids = jax.lax.broadcasted_iota(jnp.int32, (g, block_n), 0)
    onehot = (ids == batch_ref[0, 0, :][None, :]).astype(jnp.float32)
    o_ref[...] += jnp.dot(onehot, final, preferred_element_type=jnp.float32)
    cnt_ref[...] += jnp.broadcast_to(
        jnp.sum(onehot, axis=1, keepdims=True), cnt_ref.shape)

    @pl.when(pl.program_id(0) == n_blocks - 1)
    def _():
        o_ref[...] = o_ref[...] / jnp.maximum(cnt_ref[...], 1.0)


def _pick_block(total, target):
    for b in range(min(target, total), 0, -1):
        if total % b == 0:
            return b
    return total


def kernel(x, edge_index_rel0, edge_index_rel1, batch,
           W0_rel0, b0_rel0, W0_rel1, b0_rel1,
           W1_rel0, b1_rel0, W1_rel1, b1_rel1):
    f32 = jnp.float32
    n, d_in = x.shape
    d_hid = W0_rel0.shape[1]
    d_out = W1_rel0.shape[1]
    e = edge_index_rel0.shape[1]
    g = 64

    block_n = _pick_block(n, 2000)
    n_nb = n // block_n
    block_e = _pick_block(e, 4000)
    e_nb = e // block_e

    rows0 = edge_index_rel0[0].astype(jnp.int32).reshape(e_nb, 1, block_e)
    cols0 = edge_index_rel0[1].astype(jnp.int32).reshape(e_nb, 1, block_e)
    rows1 = edge_index_rel1[0].astype(jnp.int32).reshape(e_nb, 1, block_e)
    cols1 = edge_index_rel1[1].astype(jnp.int32).reshape(e_nb, 1, block_e)
    batch3 = batch.astype(jnp.int32).reshape(n_nb, 1, block_n)

    smem_idx = pl.BlockSpec((1, 1, block_e), lambda i: (i, 0, 0),
                            memory_space=pltpu.SMEM)

    deg_call = pl.pallas_call(
        functools.partial(_deg_body, block_e=block_e, n=n),
        grid=(e_nb,),
        in_specs=[smem_idx],
        out_specs=pl.BlockSpec((n, 1), lambda i: (0, 0)),
        out_shape=jax.ShapeDtypeStruct((n, 1), f32),
    )
    dinv0 = jax.lax.rsqrt(deg_call(cols0) + 1.0)
    dinv1 = jax.lax.rsqrt(deg_call(cols1) + 1.0)

    def mm_scale(xv, w, dinv):
        return pl.pallas_call(
            _mm_scale_body,
            grid=(n_nb,),
            in_specs=[
                pl.BlockSpec((block_n, d_in), lambda i: (i, 0)),
                pl.BlockSpec((d_in, d_hid), lambda i: (0, 0)),
                pl.BlockSpec((block_n, 1), lambda i: (i, 0)),
            ],
            out_specs=pl.BlockSpec((block_n, d_hid), lambda i: (i, 0)),
            out_shape=jax.ShapeDtypeStruct((n, d_hid), f32),
        )(xv, w, dinv)

    def agg(ms, rows3, cols3):
        return pl.pallas_call(
            functools.partial(_agg_body, block_e=block_e),
            grid=(e_nb,),
            in_specs=[
                pl.BlockSpec((n, d_hid), lambda i: (0, 0)),
                smem_idx,
                smem_idx,
            ],
            out_specs=pl.BlockSpec((n, d_hid), lambda i: (0, 0)),
            out_shape=jax.ShapeDtypeStruct((n, d_hid), f32),
        )(ms, rows3, cols3)

    ms00 = mm_scale(x, W0_rel0, dinv0)
    ms01 = mm_scale(x, W0_rel1, dinv1)
    agg00 = agg(ms00, rows0, cols0)
    agg01 = agg(ms01, rows1, cols1)

    bsum0 = (b0_rel0 + b0_rel1).reshape(1, d_hid)

    def mm1(dr, w):
        return pl.pallas_call(
            _mm1_body,
            grid=(n_nb,),
            in_specs=[
                pl.BlockSpec((block_n, d_hid), lambda i: (i, 0)),
                pl.BlockSpec((block_n, d_hid), lambda i: (i, 0)),
                pl.BlockSpec((block_n, 1), lambda i: (i, 0)),
                pl.BlockSpec((block_n, 1), lambda i: (i, 0)),
                pl.BlockSpec((block_n, 1), lambda i: (i, 0)),
                pl.BlockSpec((d_hid, d_out), lambda i: (0, 0)),
                pl.BlockSpec((1, d_hid), lambda i: (0, 0)),
            ],
            out_specs=pl.BlockSpec((block_n, d_out), lambda i: (i, 0)),
            out_shape=jax.ShapeDtypeStruct((n, d_out), f32),
        )(agg00, agg01, dinv0, dinv1, dr, w, bsum0)

    ms10 = mm1(dinv0, W1_rel0)
    ms11 = mm1(dinv1, W1_rel1)
    agg10 = agg(ms10, rows0, cols0)
    agg11 = agg(ms11, rows1, cols1)

    bsum1 = (b1_rel0 + b1_rel1).reshape(1, d_out)

    pooled = pl.pallas_call(
        functools.partial(_pool_body, g=g, block_n=block_n, n_blocks=n_nb),
        grid=(n_nb,),
        in_specs=[
            pl.BlockSpec((block_n, d_out), lambda i: (i, 0)),
            pl.BlockSpec((block_n, d_out), lambda i: (i, 0)),
            pl.BlockSpec((block_n, 1), lambda i: (i, 0)),
            pl.BlockSpec((block_n, 1), lambda i: (i, 0)),
            pl.BlockSpec((1, d_out), lambda i: (0, 0)),
            pl.BlockSpec((1, 1, block_n), lambda i: (i, 0, 0)),
        ],
        out_specs=pl.BlockSpec((g, d_out), lambda i: (0, 0)),
        out_shape=jax.ShapeDtypeStruct((g, d_out), f32),
        scratch_shapes=[pltpu.VMEM((g, d_out), f32)],
    )(agg10, agg11, dinv0, dinv1, bsum1, batch3)

    return pooled
